# baseline (speedup 1.0000x reference)
import jax
import jax.numpy as jnp
from jax import lax
from jax.experimental import pallas as pl
from jax.experimental.pallas import tpu as pltpu
from jax.experimental.pallas import tpu_sc as plsc

N_ROWS = 100000
IN_FEATURES = 256
OUT_FEATURES = 128
N_LEVELS = 8
SEL_W = 16
LEVEL_W = 32
NUM_WORKERS = 32
N_BLOCKS = N_ROWS // 8
CB = 28
CHUNK_ROWS = CB * 8
NCH = 14


def _body(h_hbm, out_hbm, buf0, buf1, si0, si1, so0, so1):
    c = lax.axis_index("c")
    s = lax.axis_index("s")
    w = s * 2 + c
    bstart = (N_BLOCKS * w) // NUM_WORKERS
    bend = (N_BLOCKS * (w + 1)) // NUM_WORKERS
    bufs = (buf0, buf1)
    sis = (si0, si1)
    sos = (so0, so1)

    def row0_of(k):
        return jnp.minimum(bstart + k * CB, bend - CB) * 8

    def in_copy(b, r0):
        return pltpu.make_async_copy(
            h_hbm.at[pl.ds(r0, CHUNK_ROWS), :], bufs[b], sis[b]
        )

    def out_copy(b, r0):
        return pltpu.make_async_copy(
            bufs[b].at[:, pl.ds(0, OUT_FEATURES)],
            out_hbm.at[pl.ds(r0, CHUNK_ROWS), :],
            sos[b],
        )

    for b in range(2):
        in_copy(b, row0_of(b)).start()

    def step(k, b):
        in_copy(b, row0_of(k)).wait()

        @pl.when(k >= 1)
        def _():
            out_copy(1 - b, row0_of(k - 1)).wait()

        @pl.when(jnp.logical_and(k >= 1, k + 1 < NCH))
        def _():
            in_copy(1 - b, row0_of(k + 1)).start()

        def row_body(r, cc):
            for l in range(1, N_LEVELS):
                bufs[b][r, pl.ds(SEL_W * l, SEL_W)] = bufs[b][
                    r, pl.ds(LEVEL_W * l, SEL_W)
                ]
            return cc

        lax.fori_loop(0, CHUNK_ROWS, row_body, 0, unroll=8)
        out_copy(b, row0_of(k)).start()

    def outer(j, carry):
        step(j * 2, 0)
        step(j * 2 + 1, 1)
        return carry

    lax.fori_loop(0, NCH // 2, outer, 0)
    out_copy(1, row0_of(NCH - 1)).wait()


@jax.jit
def kernel(h):
    mesh = plsc.VectorSubcoreMesh(core_axis_name="c", subcore_axis_name="s")
    return pl.kernel(
        _body,
        out_type=jax.ShapeDtypeStruct((N_ROWS, OUT_FEATURES), jnp.float32),
        mesh=mesh,
        scratch_types=[
            pltpu.VMEM((CHUNK_ROWS, IN_FEATURES), jnp.float32),
            pltpu.VMEM((CHUNK_ROWS, IN_FEATURES), jnp.float32),
            pltpu.SemaphoreType.DMA,
            pltpu.SemaphoreType.DMA,
            pltpu.SemaphoreType.DMA,
            pltpu.SemaphoreType.DMA,
        ],
    )(h)
